# baseline (speedup 1.0000x reference)
import jax
import jax.numpy as jnp
from jax import lax
from jax.experimental import pallas as pl
from jax.experimental.pallas import tpu as pltpu
from jax.experimental.pallas import tpu_sc as plsc

_V = 1000000
_D = 32
_B = 4096
_T = 200
_N = _B * _T

_TC_COLS = 16384
_Q = _TC_COLS // 4
_SC_BLK = 1024
_CH = 128

_N_TC_BLOCKS = pl.cdiv(_V, _TC_COLS)
_LINES = _N_TC_BLOCKS * _Q

_QSH = _Q.bit_length() - 1
_BSH = _TC_COLS.bit_length() - 1


def _table_to_lines(table_t):
    def body(x_ref, y_ref):
        parts = []
        for k in range(4):
            parts.append(x_ref[:, pl.ds(k * _Q, _Q)].T)
        y_ref[...] = jnp.concatenate(parts, axis=1)

    return pl.pallas_call(
        body,
        grid=(_N_TC_BLOCKS,),
        in_specs=[pl.BlockSpec((_D, _TC_COLS), lambda g: (0, g))],
        out_specs=pl.BlockSpec((_Q, 128), lambda g: (g, 0)),
        out_shape=jax.ShapeDtypeStruct((_LINES, 128), jnp.float32),
    )(table_t)


def _sc_gather(idx_r, table_r):
    mesh = plsc.VectorSubcoreMesh(core_axis_name="c", subcore_axis_name="s")

    @pl.kernel(
        out_type=jax.ShapeDtypeStruct((_T, _D // 8, _B // 128, 8, 128),
                                      jnp.float32),
        mesh=mesh,
        scratch_types=[
            pltpu.VMEM((_SC_BLK,), jnp.int32),
            pltpu.VMEM((_SC_BLK,), jnp.int32),
            pltpu.VMEM((2, _CH, 128), jnp.float32),
            pltpu.SemaphoreType.DMA,
            pltpu.SemaphoreType.DMA,
        ],
        compiler_params=pltpu.CompilerParams(
            use_tc_tiling_on_sc=True, needs_layout_passes=False),
    )
    def kern(table_hbm, idx_hbm, out_hbm, iv, qv, rows, sem0, sem1):
        sems = [sem0, sem1]

        def body(i_vmem, o_vmem):
            for r in range(8):
                @plsc.parallel_loop(0, 128, step=16, unroll=2)
                def _(c0):
                    x = i_vmem[0, r, pl.ds(c0, 16)]
                    sl = pl.ds(r * 128 + c0, 16)
                    iv[sl] = lax.bitwise_or(
                        lax.shift_left(lax.shift_right_logical(x, _BSH),
                                       _QSH),
                        lax.bitwise_and(x, _Q - 1))
                    qv[sl] = lax.shift_left(
                        lax.bitwise_and(lax.shift_right_logical(x, _QSH), 3),
                        5)

            n_ch = _SC_BLK // _CH

            def start(ch):
                return pltpu.async_copy(
                    table_hbm.at[iv.at[pl.ds(ch * _CH, _CH)]],
                    rows.at[ch % 2], sems[ch % 2])

            zero16 = jnp.zeros((16,), jnp.int32)
            iota128 = lax.iota(jnp.int32, 16) * 128

            copies = [start(0), None]
            for ch in range(n_ch):
                if ch + 1 < n_ch:
                    copies[(ch + 1) % 2] = start(ch + 1)
                copies[ch % 2].wait()
                buf = rows.at[ch % 2]

                for g8 in range(_CH // 16):
                    b0 = g8 * 16
                    q16 = qv[pl.ds(ch * _CH + b0, 16)]
                    base = (iota128 + b0 * 128) + q16
                    for dt in range(4):
                        for dr in range(8):
                            flat = base + (dt * 8 + dr)
                            vals = plsc.load_gather(buf, [zero16, flat])
                            o_vmem[0, dt, ch, dr, pl.ds(b0, 16)] = vals

        pltpu.emit_pipeline(
            body,
            grid=(_N // _SC_BLK,),
            in_specs=[pl.BlockSpec((1, _SC_BLK // 128, 128),
                                   lambda g: (g, 0, 0))],
            out_specs=[pl.BlockSpec(
                (1, 4, _SC_BLK // 128, 8, 128),
                lambda g: (g // (_B // _SC_BLK), 0, g % (_B // _SC_BLK),
                           0, 0))],
            core_axis_name=("c", "s"),
            dimension_semantics=(pltpu.PARALLEL,),
        )(idx_hbm, out_hbm)

    return kern(table_r, idx_r)


def kernel(inputs, table):
    idx_r = jnp.transpose(inputs).astype(jnp.int32).reshape(
        _N // _SC_BLK, _SC_BLK // 128, 128)
    table_r = _table_to_lines(jnp.transpose(table))
    out5 = _sc_gather(idx_r, table_r)
    z = jnp.transpose(out5, (0, 1, 3, 2, 4)).reshape(_T, _D, _B)
    return jnp.transpose(z, (2, 0, 1))


# speedup vs baseline: 1.2407x; 1.2407x over previous
import jax
import jax.numpy as jnp
from jax import lax
from jax.experimental import pallas as pl
from jax.experimental.pallas import tpu as pltpu
from jax.experimental.pallas import tpu_sc as plsc

_V = 1000000
_D = 32
_B = 4096
_T = 200
_N = _B * _T

_TC_COLS = 16384
_Q = _TC_COLS // 4
_SC_BLK = 1024
_CH = 128

_N_TC_BLOCKS = pl.cdiv(_V, _TC_COLS)
_LINES = _N_TC_BLOCKS * _Q

_QSH = _Q.bit_length() - 1
_BSH = _TC_COLS.bit_length() - 1


def _table_to_lines(table_t):
    def body(x_ref, y_ref):
        parts = []
        for k in range(4):
            parts.append(x_ref[:, pl.ds(k * _Q, _Q)].T)
        y_ref[...] = jnp.concatenate(parts, axis=1)

    return pl.pallas_call(
        body,
        grid=(_N_TC_BLOCKS,),
        in_specs=[pl.BlockSpec((_D, _TC_COLS), lambda g: (0, g))],
        out_specs=pl.BlockSpec((_Q, 128), lambda g: (g, 0)),
        out_shape=jax.ShapeDtypeStruct((_LINES, 128), jnp.float32),
    )(table_t)


def _sc_gather(idx_r, table_r):
    mesh = plsc.VectorSubcoreMesh(core_axis_name="c", subcore_axis_name="s")

    @pl.kernel(
        out_type=jax.ShapeDtypeStruct((_T, _D // 8, _B // 128, 8, 128),
                                      jnp.float32),
        mesh=mesh,
        scratch_types=[
            pltpu.VMEM((_SC_BLK,), jnp.int32),
            pltpu.VMEM((_SC_BLK,), jnp.int32),
            pltpu.VMEM((2, _CH, 128), jnp.float32),
            pltpu.SemaphoreType.DMA,
            pltpu.SemaphoreType.DMA,
        ],
        compiler_params=pltpu.CompilerParams(
            use_tc_tiling_on_sc=True, needs_layout_passes=False),
    )
    def kern(table_hbm, idx_hbm, out_hbm, iv, qv, rows, sem0, sem1):
        sems = [sem0, sem1]

        def body(i_vmem, o_vmem):
            for r in range(8):
                for c in range(8):
                    x = i_vmem[0, r, pl.ds(c * 16, 16)]
                    sl = pl.ds(r * 128 + c * 16, 16)
                    iv[sl] = lax.bitwise_or(
                        lax.shift_left(lax.shift_right_logical(x, _BSH),
                                       _QSH),
                        lax.bitwise_and(x, _Q - 1))
                    qv[sl] = lax.shift_left(
                        lax.bitwise_and(lax.shift_right_logical(x, _QSH), 3),
                        5)

            n_ch = _SC_BLK // _CH

            def start(ch):
                return pltpu.async_copy(
                    table_hbm.at[iv.at[pl.ds(ch * _CH, _CH)]],
                    rows.at[ch % 2], sems[ch % 2])

            zero16 = jnp.zeros((16,), jnp.int32)
            iota128 = lax.iota(jnp.int32, 16) * 128

            copies = [start(0), None]
            for ch in range(n_ch):
                if ch + 1 < n_ch:
                    copies[(ch + 1) % 2] = start(ch + 1)
                copies[ch % 2].wait()
                buf = rows.at[ch % 2]

                @plsc.parallel_loop(0, _CH, step=16, unroll=4)
                def _(b0):
                    base = iota128 + (b0 * 128 + qv[pl.ds(ch * _CH + b0, 16)])
                    for dt in range(4):
                        for dr in range(8):
                            flat = base + (dt * 8 + dr)
                            vals = plsc.load_gather(buf, [zero16, flat])
                            o_vmem[0, dt, ch, dr, pl.ds(b0, 16)] = vals

        pltpu.emit_pipeline(
            body,
            grid=(_N // _SC_BLK,),
            in_specs=[pl.BlockSpec((1, _SC_BLK // 128, 128),
                                   lambda g: (g, 0, 0))],
            out_specs=[pl.BlockSpec(
                (1, 4, _SC_BLK // 128, 8, 128),
                lambda g: (g // (_B // _SC_BLK), 0, g % (_B // _SC_BLK),
                           0, 0))],
            core_axis_name=("c", "s"),
            dimension_semantics=(pltpu.PARALLEL,),
        )(idx_hbm, out_hbm)

    return kern(table_r, idx_r)


def kernel(inputs, table):
    idx_r = jnp.transpose(inputs).astype(jnp.int32).reshape(
        _N // _SC_BLK, _SC_BLK // 128, 128)
    table_r = _table_to_lines(jnp.transpose(table))
    out5 = _sc_gather(idx_r, table_r)
    z = jnp.transpose(out5, (0, 1, 3, 2, 4)).reshape(_T, _D, _B)
    return jnp.transpose(z, (2, 0, 1))


# speedup vs baseline: 1.2564x; 1.0126x over previous
import jax
import jax.numpy as jnp
from jax import lax
from jax.experimental import pallas as pl
from jax.experimental.pallas import tpu as pltpu
from jax.experimental.pallas import tpu_sc as plsc

_V = 1000000
_D = 32
_B = 4096
_T = 200
_N = _B * _T

_TC_COLS = 16384
_Q = _TC_COLS // 4
_SC_BLK = 1024
_CH = 128

_N_TC_BLOCKS = pl.cdiv(_V, _TC_COLS)
_LINES = _N_TC_BLOCKS * _Q

_QSH = _Q.bit_length() - 1
_BSH = _TC_COLS.bit_length() - 1


def _table_to_lines(table_t):
    def body(x_ref, y_ref):
        parts = []
        for k in range(4):
            parts.append(x_ref[:, pl.ds(k * _Q, _Q)].T)
        y_ref[...] = jnp.concatenate(parts, axis=1)

    return pl.pallas_call(
        body,
        grid=(_N_TC_BLOCKS,),
        in_specs=[pl.BlockSpec((_D, _TC_COLS), lambda g: (0, g))],
        out_specs=pl.BlockSpec((_Q, 128), lambda g: (g, 0)),
        out_shape=jax.ShapeDtypeStruct((_LINES, 128), jnp.float32),
    )(table_t)


def _sc_gather(idx_r, table_r):
    mesh = plsc.VectorSubcoreMesh(core_axis_name="c", subcore_axis_name="s")

    @pl.kernel(
        out_type=jax.ShapeDtypeStruct((_T, _D // 8, _B // 128, 8, 128),
                                      jnp.float32),
        mesh=mesh,
        scratch_types=[
            pltpu.VMEM((_SC_BLK,), jnp.int32),
            pltpu.VMEM((_SC_BLK,), jnp.int32),
            pltpu.VMEM((2, _CH, 128), jnp.float32),
            pltpu.SemaphoreType.DMA,
            pltpu.SemaphoreType.DMA,
        ],
        compiler_params=pltpu.CompilerParams(
            use_tc_tiling_on_sc=True, needs_layout_passes=False),
    )
    def kern(table_hbm, idx_hbm, out_hbm, iv, qv, rows, sem0, sem1):
        sems = [sem0, sem1]

        def body(i_vmem, o_vmem):
            for r in range(8):
                for c in range(8):
                    x = i_vmem[0, r, pl.ds(c * 16, 16)]
                    sl = pl.ds(r * 128 + c * 16, 16)
                    iv[sl] = lax.bitwise_or(
                        lax.shift_left(lax.shift_right_logical(x, _BSH),
                                       _QSH),
                        lax.bitwise_and(x, _Q - 1))
                    qv[sl] = lax.shift_left(
                        lax.bitwise_and(lax.shift_right_logical(x, _QSH), 3),
                        5)

            n_ch = _SC_BLK // _CH

            def start(ch):
                return pltpu.async_copy(
                    table_hbm.at[iv.at[pl.ds(ch * _CH, _CH)]],
                    rows.at[ch % 2], sems[ch % 2])

            copies = [start(0), None]
            for ch in range(n_ch):
                if ch + 1 < n_ch:
                    copies[(ch + 1) % 2] = start(ch + 1)
                copies[ch % 2].wait()
                buf = rows.at[ch % 2]

                for half in range(_CH // 128):
                    btl = ch * (_CH // 128) + half

                    @plsc.parallel_loop(0, 128, step=16, unroll=8)
                    def _(b0):
                        row_ids = lax.iota(jnp.int32, 16) + (half * 128 + b0)
                        q16 = qv[pl.ds(ch * _CH + half * 128 + b0, 16)]
                        for dt in range(4):
                            for dr in range(8):
                                col_ids = q16 + (dt * 8 + dr)
                                vals = plsc.load_gather(
                                    buf, [row_ids, col_ids])
                                o_vmem[0, dt, btl, dr, pl.ds(b0, 16)] = vals

        pltpu.emit_pipeline(
            body,
            grid=(_N // _SC_BLK,),
            in_specs=[pl.BlockSpec((1, _SC_BLK // 128, 128),
                                   lambda g: (g, 0, 0))],
            out_specs=[pl.BlockSpec(
                (1, 4, _SC_BLK // 128, 8, 128),
                lambda g: (g // (_B // _SC_BLK), 0, g % (_B // _SC_BLK),
                           0, 0))],
            core_axis_name=("c", "s"),
            dimension_semantics=(pltpu.PARALLEL,),
        )(idx_hbm, out_hbm)

    return kern(table_r, idx_r)


def kernel(inputs, table):
    idx_r = jnp.transpose(inputs).astype(jnp.int32).reshape(
        _N // _SC_BLK, _SC_BLK // 128, 128)
    table_r = _table_to_lines(jnp.transpose(table))
    out5 = _sc_gather(idx_r, table_r)
    z = jnp.transpose(out5, (0, 1, 3, 2, 4)).reshape(_T, _D, _B)
    return jnp.transpose(z, (2, 0, 1))


# speedup vs baseline: 1.3245x; 1.0542x over previous
import jax
import jax.numpy as jnp
from jax import lax
from jax.experimental import pallas as pl
from jax.experimental.pallas import tpu as pltpu
from jax.experimental.pallas import tpu_sc as plsc

_V = 1000000
_D = 32
_B = 4096
_T = 200
_N = _B * _T

_TC_COLS = 16384
_Q = _TC_COLS // 4
_SC_BLK = 1024
_CH = 128

_N_TC_BLOCKS = pl.cdiv(_V, _TC_COLS)
_LINES = _N_TC_BLOCKS * _Q

_QSH = _Q.bit_length() - 1
_BSH = _TC_COLS.bit_length() - 1


def _table_to_lines(table_t):
    def body(x_ref, y_ref):
        parts = []
        for k in range(4):
            parts.append(x_ref[:, pl.ds(k * _Q, _Q)].T)
        y_ref[...] = jnp.concatenate(parts, axis=1)

    return pl.pallas_call(
        body,
        grid=(_N_TC_BLOCKS,),
        in_specs=[pl.BlockSpec((_D, _TC_COLS), lambda g: (0, g))],
        out_specs=pl.BlockSpec((_Q, 128), lambda g: (g, 0)),
        out_shape=jax.ShapeDtypeStruct((_LINES, 128), jnp.float32),
    )(table_t)


def _sc_gather(idx_r, table_r):
    mesh = plsc.VectorSubcoreMesh(core_axis_name="c", subcore_axis_name="s")

    @pl.kernel(
        out_type=jax.ShapeDtypeStruct((_T, _D // 8, _B // 128, 8, 128),
                                      jnp.float32),
        mesh=mesh,
        scratch_types=[
            pltpu.VMEM((_SC_BLK,), jnp.int32),
            pltpu.VMEM((_SC_BLK,), jnp.int32),
            pltpu.VMEM((2, _CH, 128), jnp.float32),
            pltpu.SemaphoreType.DMA,
            pltpu.SemaphoreType.DMA,
        ],
        compiler_params=pltpu.CompilerParams(
            use_tc_tiling_on_sc=True, needs_layout_passes=False),
    )
    def kern(table_hbm, idx_hbm, out_hbm, iv, qv, rows, sem0, sem1):
        sems = [sem0, sem1]

        def body(i_vmem, o_vmem):
            for r in range(8):
                for c in range(8):
                    x = i_vmem[0, r, pl.ds(c * 16, 16)]
                    sl = pl.ds(r * 128 + c * 16, 16)
                    iv[sl] = lax.bitwise_or(
                        lax.shift_left(lax.shift_right_logical(x, _BSH),
                                       _QSH),
                        lax.bitwise_and(x, _Q - 1))
                    qv[sl] = lax.shift_left(
                        lax.bitwise_and(lax.shift_right_logical(x, _QSH), 3),
                        5)

            n_ch = _SC_BLK // _CH

            def start(ch):
                return pltpu.async_copy(
                    table_hbm.at[iv.at[pl.ds(ch * _CH, _CH)]],
                    rows.at[ch % 2], sems[ch % 2])

            copies = [start(0), None]
            for ch in range(n_ch):
                if ch + 1 < n_ch:
                    copies[(ch + 1) % 2] = start(ch + 1)
                copies[ch % 2].wait()
                buf = rows.at[ch % 2]

                for half in range(_CH // 128):
                    btl = ch * (_CH // 128) + half

                    @plsc.parallel_loop(0, 128, step=16, unroll=4)
                    def _(b0):
                        row_ids = lax.iota(jnp.int32, 16) + (half * 128 + b0)
                        q16 = qv[pl.ds(ch * _CH + half * 128 + b0, 16)]
                        for dt in range(4):
                            for dr in range(8):
                                col_ids = q16 + (dt * 8 + dr)
                                vals = plsc.load_gather(
                                    buf, [row_ids, col_ids])
                                o_vmem[0, dt, btl, dr, pl.ds(b0, 16)] = vals

        pltpu.emit_pipeline(
            body,
            grid=(_N // _SC_BLK,),
            in_specs=[pl.BlockSpec((1, _SC_BLK // 128, 128),
                                   lambda g: (g, 0, 0))],
            out_specs=[pl.BlockSpec(
                (1, 4, _SC_BLK // 128, 8, 128),
                lambda g: (g // (_B // _SC_BLK), 0, g % (_B // _SC_BLK),
                           0, 0))],
            core_axis_name=("c", "s"),
            dimension_semantics=(pltpu.PARALLEL,),
        )(idx_hbm, out_hbm)

    return kern(table_r, idx_r)


def kernel(inputs, table):
    idx_r = jnp.transpose(inputs).astype(jnp.int32).reshape(
        _N // _SC_BLK, _SC_BLK // 128, 128)
    table_r = _table_to_lines(jnp.transpose(table))
    out5 = _sc_gather(idx_r, table_r)
    z = jnp.transpose(out5, (0, 1, 3, 2, 4)).reshape(_T, _D, _B)
    return jnp.transpose(z, (2, 0, 1))
